# baseline (speedup 1.0000x reference)
import functools

import jax
import jax.numpy as jnp
from jax import lax
from jax.experimental import pallas as pl
from jax.experimental.pallas import tpu as pltpu
from jax.experimental.pallas import tpu_sc as plsc

S = 200
B = 4096
D = 64
V = 1000000
S_BLK = 25
B_BLK = 1024
N_TOK = S_BLK * B_BLK
CHUNK = 256
CPR = B_BLK // CHUNK
N_CHUNK = N_TOK // CHUNK
GRP = CHUNK // 16


def _gather_fmt(w_rm, ids_t):
    mesh = plsc.VectorSubcoreMesh(core_axis_name="c", subcore_axis_name="s")

    @functools.partial(
        pl.kernel,
        out_type=jax.ShapeDtypeStruct((S, D, B), jnp.float32),
        mesh=mesh,
        scratch_types=[
            pltpu.VMEM((N_TOK,), jnp.int32),
            pltpu.VMEM((2, CHUNK, D), jnp.float32),
            pltpu.VMEM((2, D, CHUNK), jnp.float32),
            pltpu.SemaphoreType.DMA,
            pltpu.SemaphoreType.DMA,
            pltpu.SemaphoreType.DMA,
            pltpu.SemaphoreType.DMA,
        ],
        compiler_params=pltpu.CompilerParams(
            use_tc_tiling_on_sc=False, needs_layout_passes=False
        ),
    )
    def body(w_hbm, ids_hbm, out_hbm, ids_v, rows, outv, sg0, sg1, sw0, sw1):
        cid = lax.axis_index("c")
        sid = lax.axis_index("s")
        wid = sid * 2 + cid
        s0 = (wid // 4) * S_BLK
        b0 = (wid % 4) * B_BLK
        for s in range(S_BLK):
            pltpu.sync_copy(
                ids_hbm.at[s0 + s, pl.ds(b0, B_BLK)],
                ids_v.at[pl.ds(s * B_BLK, B_BLK)],
            )
        gsems = (sg0, sg1)
        wsems = (sw0, sw1)
        iota = lax.iota(jnp.int32, 16)
        row_idxs = [iota + (tg * 16) for tg in range(GRP)]

        def gather_desc(k, slot):
            return pltpu.make_async_copy(
                w_hbm.at[ids_v.at[pl.ds(k * CHUNK, CHUNK)]],
                rows.at[slot],
                gsems[slot],
            )

        def write_desc(k, slot):
            s_idx = s0 + k // CPR
            boff = b0 + lax.rem(k, CPR) * CHUNK
            return pltpu.make_async_copy(
                outv.at[slot],
                out_hbm.at[s_idx, :, pl.ds(boff, CHUNK)],
                wsems[slot],
            )

        def do_chunk(j, k, slot):
            gather_desc(k, slot).wait()
            @pl.when(j > 0)
            def _drain():
                write_desc(k - 2, slot).wait()

            rows_c = rows.at[slot]
            out_c = outv.at[slot]
            for d in range(D):
                col = jnp.full((16,), d, jnp.int32)
                for tg in range(GRP):
                    vals = plsc.load_gather(rows_c, [row_idxs[tg], col])
                    out_c[d, pl.ds(tg * 16, 16)] = vals
            write_desc(k, slot).start()

        def pair_body(j, carry):
            k0 = j * 2
            gather_desc(k0 + 1, 1).start()
            do_chunk(j, k0, 0)
            @pl.when(k0 + 2 < N_CHUNK)
            def _pref():
                gather_desc(k0 + 2, 0).start()

            do_chunk(j, k0 + 1, 1)
            return carry

        gather_desc(0, 0).start()
        lax.fori_loop(0, N_CHUNK // 2, pair_body, 0)
        write_desc(N_CHUNK - 2, 0).wait()
        write_desc(N_CHUNK - 1, 1).wait()

    return body(w_rm, ids_t)


def kernel(token_ids, weight):
    ids_t = token_ids.T.astype(jnp.int32)
    out_phys = _gather_fmt(weight, ids_t)
    return out_phys.transpose(2, 0, 1)


# speedup vs baseline: 1.2238x; 1.2238x over previous
import functools

import jax
import jax.numpy as jnp
from jax import lax
from jax.experimental import pallas as pl
from jax.experimental.pallas import tpu as pltpu
from jax.experimental.pallas import tpu_sc as plsc

S = 200
B = 4096
D = 64
V = 1000000
S_BLK = 25
B_BLK = 1024
N_TOK = S_BLK * B_BLK
CHUNK = 256
CPR = B_BLK // CHUNK
N_CHUNK = N_TOK // CHUNK
GRP = CHUNK // 16


def _gather_fmt(w_rm, ids_t):
    mesh = plsc.VectorSubcoreMesh(core_axis_name="c", subcore_axis_name="s")

    @functools.partial(
        pl.kernel,
        out_type=jax.ShapeDtypeStruct((S, D, B), jnp.float32),
        mesh=mesh,
        scratch_types=[
            pltpu.VMEM((N_TOK,), jnp.int32),
            pltpu.VMEM((2, CHUNK, D), jnp.float32),
            pltpu.VMEM((2, D, CHUNK), jnp.float32),
            pltpu.SemaphoreType.DMA,
            pltpu.SemaphoreType.DMA,
            pltpu.SemaphoreType.DMA,
            pltpu.SemaphoreType.DMA,
        ],
        compiler_params=pltpu.CompilerParams(
            use_tc_tiling_on_sc=False, needs_layout_passes=False
        ),
    )
    def body(w_hbm, ids_hbm, out_hbm, ids_v, rows, outv, sg0, sg1, sw0, sw1):
        cid = lax.axis_index("c")
        sid = lax.axis_index("s")
        wid = sid * 2 + cid
        s0 = (wid // 4) * S_BLK
        b0 = (wid % 4) * B_BLK
        for s in range(S_BLK):
            pltpu.sync_copy(
                ids_hbm.at[s0 + s, pl.ds(b0, B_BLK)],
                ids_v.at[pl.ds(s * B_BLK, B_BLK)],
            )
        gsems = (sg0, sg1)
        wsems = (sw0, sw1)
        iota = lax.iota(jnp.int32, 16)
        row_idxs = [iota + (tg * 16) for tg in range(GRP)]

        def gather_desc(k, slot):
            return pltpu.make_async_copy(
                w_hbm.at[ids_v.at[pl.ds(k * CHUNK, CHUNK)]],
                rows.at[slot],
                gsems[slot],
            )

        def write_desc(k, slot):
            s_idx = s0 + k // CPR
            boff = b0 + lax.rem(k, CPR) * CHUNK
            return pltpu.make_async_copy(
                outv.at[slot],
                out_hbm.at[s_idx, :, pl.ds(boff, CHUNK)],
                wsems[slot],
            )

        def do_chunk(j, k, slot):
            gather_desc(k, slot).wait()
            @pl.when(j > 0)
            def _drain():
                write_desc(k - 2, slot).wait()

            rows_c = rows.at[slot]
            out_c = outv.at[slot]
            for d in range(D):
                col = jnp.full((16,), d, jnp.int32)
                vals = [
                    plsc.load_gather(rows_c, [row_idxs[tg], col])
                    for tg in range(GRP)
                ]
                for tg in range(GRP):
                    out_c[d, pl.ds(tg * 16, 16)] = vals[tg]
            write_desc(k, slot).start()

        def pair_body(j, carry):
            k0 = j * 2
            gather_desc(k0 + 1, 1).start()
            do_chunk(j, k0, 0)
            @pl.when(k0 + 2 < N_CHUNK)
            def _pref():
                gather_desc(k0 + 2, 0).start()

            do_chunk(j, k0 + 1, 1)
            return carry

        gather_desc(0, 0).start()
        lax.fori_loop(0, N_CHUNK // 2, pair_body, 0)
        write_desc(N_CHUNK - 2, 0).wait()
        write_desc(N_CHUNK - 1, 1).wait()

    return body(w_rm, ids_t)


def kernel(token_ids, weight):
    ids_t = token_ids.T.astype(jnp.int32)
    out_phys = _gather_fmt(weight, ids_t)
    return out_phys.transpose(2, 0, 1)


# speedup vs baseline: 2.0394x; 1.6665x over previous
import functools

import jax
import jax.numpy as jnp
from jax import lax
from jax.experimental import pallas as pl
from jax.experimental.pallas import tpu as pltpu
from jax.experimental.pallas import tpu_sc as plsc

S = 200
B = 4096
D = 64
V = 1000000
S_BLK = 25
B_BLK = 1024
N_TOK = S_BLK * B_BLK
CHUNK = 256
CPR = B_BLK // CHUNK
N_CHUNK = N_TOK // CHUNK
GRP = CHUNK // 16


def _gather_fmt(w_rm, ids_t):
    mesh = plsc.VectorSubcoreMesh(core_axis_name="c", subcore_axis_name="s")

    @functools.partial(
        pl.kernel,
        out_type=jax.ShapeDtypeStruct((S, D, B), jnp.float32),
        mesh=mesh,
        scratch_types=[
            pltpu.VMEM((N_TOK,), jnp.int32),
            pltpu.VMEM((2, CHUNK, D), jnp.float32),
            pltpu.VMEM((2, D, CHUNK + 1), jnp.float32),
            pltpu.SemaphoreType.DMA,
            pltpu.SemaphoreType.DMA,
            pltpu.SemaphoreType.DMA,
            pltpu.SemaphoreType.DMA,
        ],
        compiler_params=pltpu.CompilerParams(
            use_tc_tiling_on_sc=False, needs_layout_passes=False
        ),
    )
    def body(w_hbm, ids_hbm, out_hbm, ids_v, rows, outv, sg0, sg1, sw0, sw1):
        cid = lax.axis_index("c")
        sid = lax.axis_index("s")
        wid = sid * 2 + cid
        s0 = (wid // 4) * S_BLK
        b0 = (wid % 4) * B_BLK
        for s in range(S_BLK):
            pltpu.sync_copy(
                ids_hbm.at[s0 + s, pl.ds(b0, B_BLK)],
                ids_v.at[pl.ds(s * B_BLK, B_BLK)],
            )
        gsems = (sg0, sg1)
        wsems = (sw0, sw1)
        iota = lax.iota(jnp.int32, 16)
        row_idxs = [iota + (tg * 16) for tg in range(GRP)]

        def gather_desc(k, slot):
            return pltpu.make_async_copy(
                w_hbm.at[ids_v.at[pl.ds(k * CHUNK, CHUNK)]],
                rows.at[slot],
                gsems[slot],
            )

        def write_desc(k, slot):
            s_idx = s0 + k // CPR
            boff = b0 + lax.rem(k, CPR) * CHUNK
            return pltpu.make_async_copy(
                outv.at[slot, :, pl.ds(0, CHUNK)],
                out_hbm.at[s_idx, :, pl.ds(boff, CHUNK)],
                wsems[slot],
            )

        def do_chunk(j, k, slot):
            gather_desc(k, slot).wait()
            @pl.when(j > 0)
            def _drain():
                write_desc(k - 2, slot).wait()

            rows_c = rows.at[slot]
            out_c = outv.at[slot]
            for t in range(CHUNK):
                colt = jnp.full((16,), t, jnp.int32)
                for dg in range(D // 16):
                    vals = rows_c[t, pl.ds(dg * 16, 16)]
                    plsc.store_scatter(out_c, [row_idxs[dg], colt], vals)
            write_desc(k, slot).start()

        def pair_body(j, carry):
            k0 = j * 2
            gather_desc(k0 + 1, 1).start()
            do_chunk(j, k0, 0)
            @pl.when(k0 + 2 < N_CHUNK)
            def _pref():
                gather_desc(k0 + 2, 0).start()

            do_chunk(j, k0 + 1, 1)
            return carry

        gather_desc(0, 0).start()
        lax.fori_loop(0, N_CHUNK // 2, pair_body, 0)
        write_desc(N_CHUNK - 2, 0).wait()
        write_desc(N_CHUNK - 1, 1).wait()

    return body(w_rm, ids_t)


def kernel(token_ids, weight):
    ids_t = token_ids.T.astype(jnp.int32)
    out_phys = _gather_fmt(weight, ids_t)
    return out_phys.transpose(2, 0, 1)
